# Initial kernel scaffold; baseline (speedup 1.0000x reference)
import jax
import jax.numpy as jnp
from jax.experimental import pallas as pl


def kernel(x, Wr, W1, b1, W2, b2):
    raise NotImplementedError("write your pallas kernel here")


# speedup vs baseline: 2.5138x; 2.5138x over previous
import functools
import math

import jax
import jax.numpy as jnp
from jax.experimental import pallas as pl

E = 8
K = 2
D = 768
H = 3072
S = 2048

_INV_SQRT2 = 1.0 / math.sqrt(2.0)


def _gelu_exact(x):
    return 0.5 * x * (1.0 + jax.lax.erf(x * _INV_SQRT2))


def _router_body(x_ref, wr_ref, probs_ref, sel_ref, rw_ref, disp_ref):
    x = x_ref[...]
    wr = wr_ref[...]
    logits = jax.lax.dot_general(wr, x, (((1,), (1,)), ((), ())),
                                 preferred_element_type=jnp.float32)
    m = jnp.max(logits, axis=0, keepdims=True)
    ex = jnp.exp(logits - m)
    probs = ex / jnp.sum(ex, axis=0, keepdims=True)
    probs_ref[...] = probs

    iota_e = jax.lax.broadcasted_iota(jnp.int32, (E, S), 0)
    m0 = jnp.max(probs, axis=0, keepdims=True)
    a0 = jnp.min(jnp.where(probs == m0, iota_e, E), axis=0, keepdims=True)
    masked = jnp.where(iota_e == a0, -jnp.inf, probs)
    m1 = jnp.max(masked, axis=0, keepdims=True)
    a1 = jnp.min(jnp.where(masked == m1, iota_e, E), axis=0, keepdims=True)

    denom = m0 + m1
    w0 = m0 / denom
    w1 = m1 / denom
    sel_ref[...] = jnp.concatenate([a0, a1], axis=0)
    rw_ref[...] = jnp.concatenate([w0, w1], axis=0)
    disp_ref[...] = (jnp.where(iota_e == a0, w0, 0.0)
                     + jnp.where(iota_e == a1, w1, 0.0))


def _ffn_body(x_ref, w1_ref, b1_ref, w2_ref, b2_ref, w_ref, o_ref):
    e = pl.program_id(1)
    h = pl.program_id(2)

    @pl.when((e == 0) & (h == 0))
    def _():
        o_ref[...] = jnp.zeros_like(o_ref)

    xb = x_ref[...]
    hpre = jax.lax.dot_general(xb, w1_ref[0], (((1,), (1,)), ((), ())),
                               preferred_element_type=jnp.float32)
    hpre = hpre + b1_ref[0]
    hact = _gelu_exact(hpre)
    contrib = jax.lax.dot_general(hact, w2_ref[0], (((1,), (1,)), ((), ())),
                                  preferred_element_type=jnp.float32)
    wcol = w_ref[0]
    o_ref[...] += wcol * contrib

    @pl.when(h == 0)
    def _():
        o_ref[...] += wcol * b2_ref[0]


def kernel(x, Wr, W1, b1, W2, b2):
    B = x.shape[0]
    x2 = x.reshape(B * S, D)

    probsT, selT, rwT, dispT = pl.pallas_call(
        _router_body,
        out_shape=[
            jax.ShapeDtypeStruct((E, S), jnp.float32),
            jax.ShapeDtypeStruct((K, S), jnp.int32),
            jax.ShapeDtypeStruct((K, S), jnp.float32),
            jax.ShapeDtypeStruct((E, S), jnp.float32),
        ],
    )(x2, Wr)

    disp3 = dispT.reshape(E, S, 1)

    TB = 512
    HB = 1024
    nt, nh = S // TB, H // HB
    out = pl.pallas_call(
        _ffn_body,
        grid=(nt, E, nh),
        in_specs=[
            pl.BlockSpec((TB, D), lambda t, e, h: (t, 0)),
            pl.BlockSpec((1, HB, D), lambda t, e, h: (e, h, 0)),
            pl.BlockSpec((1, 1, HB), lambda t, e, h: (e, 0, h)),
            pl.BlockSpec((1, D, HB), lambda t, e, h: (e, 0, h)),
            pl.BlockSpec((1, 1, D), lambda t, e, h: (e, 0, 0)),
            pl.BlockSpec((1, TB, 1), lambda t, e, h: (e, t, 0)),
        ],
        out_specs=pl.BlockSpec((TB, D), lambda t, e, h: (t, 0)),
        out_shape=jax.ShapeDtypeStruct((B * S, D), jnp.float32),
    )(x2, W1, b1.reshape(E, 1, H), W2, b2.reshape(E, 1, D), disp3)

    expert_outputs = out.reshape(B, S, D)
    routing_probs = probsT.T.reshape(B, S, E)
    selected_experts = selT.T.reshape(B, S, K)
    routing_weights = rwT.T.reshape(B, S, K)
    return (expert_outputs, routing_probs, selected_experts, routing_weights)
